# baseline (speedup 1.0000x reference)
import functools

import jax
import jax.numpy as jnp
from jax import lax
from jax.experimental import pallas as pl
from jax.experimental.pallas import tpu as pltpu
from jax.experimental.pallas import tpu_sc as plsc

B_ = 64
F_ = 8192
K_ = 4
CO_ = 16
F2 = F_ // 2
ROWS = 256
CONVB = 8


def _topk_body(d_ref, idx_ref):
    d = d_ref[...]
    iota = lax.broadcasted_iota(jnp.int32, (ROWS, F_), 1)
    big = jnp.int32(2 ** 30)
    inf = jnp.float32(jnp.inf)
    for t in range(K_):
        m = jnp.min(d, axis=1, keepdims=True)
        im = jnp.min(jnp.where(d == m, iota, big), axis=1)
        idx_ref[:, t] = im
        if t < K_ - 1:
            d = jnp.where(iota == im[:, None], inf, d)


def _topk_half(d2, half):
    base = half * (F2 // ROWS)
    return pl.pallas_call(
        _topk_body,
        grid=(F2 // ROWS,),
        in_specs=[pl.BlockSpec((ROWS, F_), lambda i: (i + base, 0))],
        out_specs=pl.BlockSpec((ROWS, K_), lambda i: (i, 0)),
        out_shape=jax.ShapeDtypeStruct((F2, K_), jnp.int32),
    )(d2)


def _sc_gather(x2, idx_kf):
    mesh = plsc.VectorSubcoreMesh(core_axis_name="c", subcore_axis_name="s")

    @functools.partial(
        pl.kernel,
        out_type=jax.ShapeDtypeStruct((B_, K_, F2), jnp.float32),
        mesh=mesh,
        scratch_types=[
            pltpu.VMEM((K_, F2), jnp.int32),
            pltpu.VMEM((F_,), jnp.float32),
            pltpu.VMEM((K_, F2), jnp.float32),
        ],
        compiler_params=pltpu.CompilerParams(needs_layout_passes=False),
    )
    def k(x_hbm, idx_hbm, g_hbm, idx_v, row_v, out_v):
        wid = lax.axis_index("s") * 2 + lax.axis_index("c")
        pltpu.sync_copy(idx_hbm, idx_v)
        for p in range(2):
            b = p * 32 + wid
            pltpu.sync_copy(x_hbm.at[b], row_v)

            for kk in range(K_):
                @plsc.parallel_loop(0, F2 // 128, unroll=8)
                def _(j):
                    base = j * 128
                    for c in range(8):
                        off = base + c * 16
                        iv = idx_v[kk, pl.ds(off, 16)]
                        out_v[kk, pl.ds(off, 16)] = plsc.load_gather(
                            row_v, [iv])

            pltpu.sync_copy(out_v, g_hbm.at[b])

    return k(x2, idx_kf)


def _conv_body(ga_ref, gb_ref, w_ref, b_ref, o_ref):
    w = w_ref[...]
    bb = b_ref[...]
    for bi in range(CONVB):
        ya = lax.dot_general(w, ga_ref[bi], (((1,), (0,)), ((), ())),
                             preferred_element_type=jnp.float32)
        yb = lax.dot_general(w, gb_ref[bi], (((1,), (0,)), ((), ())),
                             preferred_element_type=jnp.float32)
        o_ref[bi] = jnp.maximum(
            jnp.concatenate([ya, yb], axis=1) + bb, 0.0)


def _conv(ga, gb, w, b2):
    g_spec = pl.BlockSpec((CONVB, K_, F2), lambda i: (i, 0, 0))
    return pl.pallas_call(
        _conv_body,
        grid=(B_ // CONVB,),
        in_specs=[
            g_spec, g_spec,
            pl.BlockSpec((CO_, K_), lambda i: (0, 0)),
            pl.BlockSpec((CO_, 1), lambda i: (0, 0)),
        ],
        out_specs=pl.BlockSpec((CONVB, CO_, F_), lambda i: (i, 0, 0)),
        out_shape=jax.ShapeDtypeStruct((B_, CO_, F_), jnp.float32),
    )(ga, gb, w, b2)


def kernel(X, Coord, distances, W, b):
    d2 = distances[0]
    x2 = X[:, 0, :]
    c2 = Coord[:, 0, :]
    idx_a = _topk_half(d2, 0)
    ia = idx_a.T
    gxa = _sc_gather(x2, ia)
    gca = _sc_gather(c2, ia)
    idx_b = _topk_half(d2, 1)
    ib = idx_b.T
    gxb = _sc_gather(x2, ib)
    gcb = _sc_gather(c2, ib)
    w2 = W[:, 0, :]
    b2 = b.reshape(CO_, 1)
    ox = _conv(gxa, gxb, w2, b2)
    oc = _conv(gca, gcb, w2, b2)
    return (ox, oc)


# speedup vs baseline: 1.0513x; 1.0513x over previous
import functools

import jax
import jax.numpy as jnp
from jax import lax
from jax.experimental import pallas as pl
from jax.experimental.pallas import tpu as pltpu
from jax.experimental.pallas import tpu_sc as plsc

B_ = 64
F_ = 8192
K_ = 4
CO_ = 16
F2 = F_ // 2
ROWS = 256
CONVB = 8


def _topk_body(d_ref, idx_ref):
    d = d_ref[...]
    iota = lax.broadcasted_iota(jnp.int32, (ROWS, F_), 1)
    big = jnp.int32(2 ** 30)
    inf = jnp.float32(jnp.inf)
    for t in range(K_):
        m = jnp.min(d, axis=1, keepdims=True)
        im = jnp.min(jnp.where(d == m, iota, big), axis=1)
        idx_ref[t, :] = im
        if t < K_ - 1:
            d = jnp.where(iota == im[:, None], inf, d)


def _topk_half(d2, half):
    base = half * (F2 // ROWS)
    return pl.pallas_call(
        _topk_body,
        grid=(F2 // ROWS,),
        in_specs=[pl.BlockSpec((ROWS, F_), lambda i: (i + base, 0))],
        out_specs=pl.BlockSpec((K_, ROWS), lambda i: (0, i)),
        out_shape=jax.ShapeDtypeStruct((K_, F2), jnp.int32),
    )(d2)


def _sc_gather(x2, c2, idx_kf):
    mesh = plsc.VectorSubcoreMesh(core_axis_name="c", subcore_axis_name="s")

    @functools.partial(
        pl.kernel,
        out_type=[jax.ShapeDtypeStruct((B_, K_, F2), jnp.float32)] * 2,
        mesh=mesh,
        scratch_types=[
            pltpu.VMEM((K_, F2), jnp.int32),
            pltpu.VMEM((F_,), jnp.float32),
            pltpu.VMEM((K_, F2), jnp.float32),
        ],
        compiler_params=pltpu.CompilerParams(needs_layout_passes=False),
    )
    def k(x_hbm, c_hbm, idx_hbm, gx_hbm, gc_hbm, idx_v, row_v, out_v):
        wid = lax.axis_index("s") * 2 + lax.axis_index("c")
        pltpu.sync_copy(idx_hbm, idx_v)
        for p in range(4):
            pid = p * 32 + wid
            b = pid % B_
            src = x_hbm if p < 2 else c_hbm
            dst = gx_hbm if p < 2 else gc_hbm
            pltpu.sync_copy(src.at[b], row_v)

            for kk in range(K_):
                @plsc.parallel_loop(0, F2 // 128, unroll=8)
                def _(j):
                    base = j * 128
                    for c in range(8):
                        off = base + c * 16
                        iv = idx_v[kk, pl.ds(off, 16)]
                        out_v[kk, pl.ds(off, 16)] = plsc.load_gather(
                            row_v, [iv])

            pltpu.sync_copy(out_v, dst.at[b])

    return k(x2, c2, idx_kf)


def _conv_body(gxa_ref, gxb_ref, gca_ref, gcb_ref, w_ref, b_ref,
               ox_ref, oc_ref):
    w = w_ref[...]
    bb = b_ref[...]
    for bi in range(CONVB):
        for (ga_ref, gb_ref, o_ref) in ((gxa_ref, gxb_ref, ox_ref),
                                        (gca_ref, gcb_ref, oc_ref)):
            ya = lax.dot_general(w, ga_ref[bi], (((1,), (0,)), ((), ())),
                                 preferred_element_type=jnp.float32)
            yb = lax.dot_general(w, gb_ref[bi], (((1,), (0,)), ((), ())),
                                 preferred_element_type=jnp.float32)
            o_ref[bi] = jnp.maximum(
                jnp.concatenate([ya, yb], axis=1) + bb, 0.0)


def _conv(gxa, gxb, gca, gcb, w, b2):
    out_sds = jax.ShapeDtypeStruct((B_, CO_, F_), jnp.float32)
    g_spec = pl.BlockSpec((CONVB, K_, F2), lambda i: (i, 0, 0))
    o_spec = pl.BlockSpec((CONVB, CO_, F_), lambda i: (i, 0, 0))
    return pl.pallas_call(
        _conv_body,
        grid=(B_ // CONVB,),
        in_specs=[
            g_spec, g_spec, g_spec, g_spec,
            pl.BlockSpec((CO_, K_), lambda i: (0, 0)),
            pl.BlockSpec((CO_, 1), lambda i: (0, 0)),
        ],
        out_specs=[o_spec, o_spec],
        out_shape=[out_sds, out_sds],
    )(gxa, gxb, gca, gcb, w, b2)


def kernel(X, Coord, distances, W, b):
    d2 = distances[0]
    x2 = X[:, 0, :]
    c2 = Coord[:, 0, :]
    idx_a = _topk_half(d2, 0)
    gxa, gca = _sc_gather(x2, c2, idx_a)
    idx_b = _topk_half(d2, 1)
    gxb, gcb = _sc_gather(x2, c2, idx_b)
    w2 = W[:, 0, :]
    b2 = b.reshape(CO_, 1)
    ox, oc = _conv(gxa, gxb, gca, gcb, w2, b2)
    return (ox, oc)


# speedup vs baseline: 1.0671x; 1.0150x over previous
import functools

import jax
import jax.numpy as jnp
from jax import lax
from jax.experimental import pallas as pl
from jax.experimental.pallas import tpu as pltpu
from jax.experimental.pallas import tpu_sc as plsc

B_ = 64
F_ = 8192
K_ = 4
CO_ = 16
F2 = F_ // 2
ROWS = 256
CONVB = 8


def _topk_body(d_ref, idx_ref):
    d = d_ref[...]
    iota = lax.broadcasted_iota(jnp.int32, (ROWS, F_), 1)
    big = jnp.int32(2 ** 30)
    inf = jnp.float32(jnp.inf)
    for t in range(K_):
        m = jnp.min(d, axis=1, keepdims=True)
        im = jnp.min(jnp.where(d == m, iota, big), axis=1)
        idx_ref[:, t] = im
        if t < K_ - 1:
            d = jnp.where(iota == im[:, None], inf, d)


def _topk_half(d2, half):
    base = half * (F2 // ROWS)
    return pl.pallas_call(
        _topk_body,
        grid=(F2 // ROWS,),
        in_specs=[pl.BlockSpec((ROWS, F_), lambda i: (i + base, 0))],
        out_specs=pl.BlockSpec((ROWS, K_), lambda i: (i, 0)),
        out_shape=jax.ShapeDtypeStruct((F2, K_), jnp.int32),
    )(d2)


def _sc_gather(x2, c2, idx_kf):
    mesh = plsc.VectorSubcoreMesh(core_axis_name="c", subcore_axis_name="s")

    @functools.partial(
        pl.kernel,
        out_type=[jax.ShapeDtypeStruct((B_, K_, F2), jnp.float32)] * 2,
        mesh=mesh,
        scratch_types=[
            pltpu.VMEM((K_, F2), jnp.int32),
            pltpu.VMEM((F_,), jnp.float32),
            pltpu.VMEM((K_, F2), jnp.float32),
        ],
        compiler_params=pltpu.CompilerParams(needs_layout_passes=False),
    )
    def k(x_hbm, c_hbm, idx_hbm, gx_hbm, gc_hbm, idx_v, row_v, out_v):
        wid = lax.axis_index("s") * 2 + lax.axis_index("c")
        pltpu.sync_copy(idx_hbm, idx_v)
        for p in range(4):
            pid = p * 32 + wid
            b = pid % B_
            src = x_hbm if p < 2 else c_hbm
            dst = gx_hbm if p < 2 else gc_hbm
            pltpu.sync_copy(src.at[b], row_v)

            for kk in range(K_):
                @plsc.parallel_loop(0, F2 // 128, unroll=8)
                def _(j):
                    base = j * 128
                    for c in range(8):
                        off = base + c * 16
                        iv = idx_v[kk, pl.ds(off, 16)]
                        out_v[kk, pl.ds(off, 16)] = plsc.load_gather(
                            row_v, [iv])

            pltpu.sync_copy(out_v, dst.at[b])

    return k(x2, c2, idx_kf)


def _conv_body(gxa_ref, gxb_ref, gca_ref, gcb_ref, w_ref, b_ref,
               ox_ref, oc_ref):
    w = w_ref[...]
    bb = b_ref[...]
    for bi in range(CONVB):
        for (ga_ref, gb_ref, o_ref) in ((gxa_ref, gxb_ref, ox_ref),
                                        (gca_ref, gcb_ref, oc_ref)):
            ya = lax.dot_general(w, ga_ref[bi], (((1,), (0,)), ((), ())),
                                 preferred_element_type=jnp.float32)
            yb = lax.dot_general(w, gb_ref[bi], (((1,), (0,)), ((), ())),
                                 preferred_element_type=jnp.float32)
            o_ref[bi] = jnp.maximum(
                jnp.concatenate([ya, yb], axis=1) + bb, 0.0)


def _conv(gxa, gxb, gca, gcb, w, b2):
    out_sds = jax.ShapeDtypeStruct((B_, CO_, F_), jnp.float32)
    g_spec = pl.BlockSpec((CONVB, K_, F2), lambda i: (i, 0, 0))
    o_spec = pl.BlockSpec((CONVB, CO_, F_), lambda i: (i, 0, 0))
    return pl.pallas_call(
        _conv_body,
        grid=(B_ // CONVB,),
        in_specs=[
            g_spec, g_spec, g_spec, g_spec,
            pl.BlockSpec((CO_, K_), lambda i: (0, 0)),
            pl.BlockSpec((CO_, 1), lambda i: (0, 0)),
        ],
        out_specs=[o_spec, o_spec],
        out_shape=[out_sds, out_sds],
    )(gxa, gxb, gca, gcb, w, b2)


def kernel(X, Coord, distances, W, b):
    d2 = distances[0]
    x2 = X[:, 0, :]
    c2 = Coord[:, 0, :]
    idx_a = _topk_half(d2, 0)
    gxa, gca = _sc_gather(x2, c2, idx_a.T)
    idx_b = _topk_half(d2, 1)
    gxb, gcb = _sc_gather(x2, c2, idx_b.T)
    w2 = W[:, 0, :]
    b2 = b.reshape(CO_, 1)
    ox, oc = _conv(gxa, gxb, gca, gcb, w2, b2)
    return (ox, oc)
